# baseline (speedup 1.0000x reference)
import functools

import jax
import jax.numpy as jnp
from jax import lax
from jax.experimental import pallas as pl
from jax.experimental.pallas import tpu as pltpu
from jax.experimental.pallas import tpu_sc as plsc

N_NODES = 10000
D = 128
E = 320000

NC = 2
NS = 16
CH = 128
CPT = 80
E_PAD = NC * NS * CPT * CH
HB = 40
N_PAD = 10240
RPT = N_PAD // NS
BB = 64
NBB = RPT // BB


def _make_agg():
    mesh = plsc.VectorSubcoreMesh(core_axis_name="c", subcore_axis_name="s")

    @functools.partial(
        pl.kernel,
        mesh=mesh,
        out_type=[
            jax.ShapeDtypeStruct((NC, N_PAD, D), jnp.float32),
            jax.ShapeDtypeStruct((NC, N_PAD, D), jnp.float32),
        ],
        scratch_types=[
            pltpu.VMEM((HB, CH), jnp.int32),
            pltpu.VMEM((HB, CH), jnp.int32),
            pltpu.VMEM((CH, D), jnp.float32),
            pltpu.VMEM((BB, D), jnp.float32),
            pltpu.VMEM((RPT,), jnp.int32),
            pltpu.VMEM((BB,), jnp.int32),
            pltpu.VMEM_SHARED((N_PAD, D), jnp.float32),
            pltpu.SemaphoreType.DMA,
        ],
    )
    def agg(src_hbm, dst_hbm, x_hbm, ones_hbm, zf_hbm, seq_hbm,
            psum_hbm, pdeg_hbm,
            src_v, dst_v, rows_v, buf_v, seq_v, seq1_v,
            sum_sh, sem):
        cid = lax.axis_index("c")
        sid = lax.axis_index("s")
        nb = sid * RPT

        pltpu.sync_copy(zf_hbm, buf_v)
        pltpu.sync_copy(seq_hbm.at[pl.ds(nb, RPT)], seq_v)

        def zinit(t, carry):
            pltpu.sync_copy(seq_hbm.at[pl.ds(nb + t * BB, BB)], seq1_v)
            pltpu.sync_copy(buf_v, sum_sh.at[seq1_v])
            return carry

        lax.fori_loop(0, NBB, zinit, 0)
        plsc.subcore_barrier()

        rowbase = (cid * NS + sid) * CPT

        for h in range(CPT // HB):
            pltpu.sync_copy(src_hbm.at[pl.ds(rowbase + h * HB, HB)], src_v)
            pltpu.sync_copy(dst_hbm.at[pl.ds(rowbase + h * HB, HB)], dst_v)

            def body(k, carry):
                pltpu.async_copy(x_hbm.at[src_v.at[k]], rows_v, sem).wait()
                pltpu.sync_copy(rows_v, sum_sh.at[dst_v.at[k]], add=True)
                return carry

            lax.fori_loop(0, HB, body, 0)
        plsc.subcore_barrier()

        def wout_sum(t, carry):
            r0 = nb + t * BB
            idx = seq_v.at[pl.ds(t * BB, BB)]
            pltpu.async_copy(sum_sh.at[idx], buf_v, sem).wait()
            pltpu.sync_copy(buf_v, psum_hbm.at[cid, pl.ds(r0, BB)])
            return carry

        lax.fori_loop(0, NBB, wout_sum, 0)
        plsc.subcore_barrier()

        pltpu.sync_copy(zf_hbm, buf_v)
        lax.fori_loop(0, NBB, zinit, 0)
        pltpu.sync_copy(ones_hbm, rows_v)
        plsc.subcore_barrier()

        for h in range(CPT // HB):
            pltpu.sync_copy(dst_hbm.at[pl.ds(rowbase + h * HB, HB)], dst_v)

            def body2(k, carry):
                pltpu.sync_copy(rows_v, sum_sh.at[dst_v.at[k]], add=True)
                return carry

            lax.fori_loop(0, HB, body2, 0)
        plsc.subcore_barrier()

        def wout_deg(t, carry):
            r0 = nb + t * BB
            idx = seq_v.at[pl.ds(t * BB, BB)]
            pltpu.async_copy(sum_sh.at[idx], buf_v, sem).wait()
            pltpu.sync_copy(buf_v, pdeg_hbm.at[cid, pl.ds(r0, BB)])
            return carry

        lax.fori_loop(0, NBB, wout_deg, 0)

    return agg


_AGG = _make_agg()

BN = 1000


def _tc_body(x_ref, ps_ref, pd_ref, wst_ref, wnt_ref, b_ref, o_ref):
    s = ps_ref[0] + ps_ref[1]
    deg = pd_ref[0, :, :1] + pd_ref[1, :, :1]
    h = s / jnp.maximum(deg, 1.0)
    acc = jnp.dot(x_ref[...], wst_ref[...], preferred_element_type=jnp.float32)
    acc = acc + jnp.dot(h, wnt_ref[...], preferred_element_type=jnp.float32)
    o_ref[...] = acc + b_ref[...]


def _combine(x, psum, pdeg, wst, wnt, b):
    return pl.pallas_call(
        _tc_body,
        grid=(N_NODES // BN,),
        in_specs=[
            pl.BlockSpec((BN, D), lambda i: (i, 0)),
            pl.BlockSpec((NC, BN, D), lambda i: (0, i, 0)),
            pl.BlockSpec((NC, BN, D), lambda i: (0, i, 0)),
            pl.BlockSpec((D, D), lambda i: (0, 0)),
            pl.BlockSpec((D, D), lambda i: (0, 0)),
            pl.BlockSpec((1, D), lambda i: (0, 0)),
        ],
        out_specs=pl.BlockSpec((BN, D), lambda i: (i, 0)),
        out_shape=jax.ShapeDtypeStruct((N_NODES, D), jnp.float32),
    )(x, psum, pdeg, wst, wnt, b)


def kernel(act_flag, x, edge_index, W_self, b_self, W_neigh, b_neigh):
    src = edge_index[0].astype(jnp.int32)
    dst = edge_index[1].astype(jnp.int32)
    npad = E_PAD - E
    src2 = jnp.concatenate([src, jnp.zeros((npad,), jnp.int32)]).reshape(-1, CH)
    dst2 = jnp.concatenate(
        [dst, jnp.full((npad,), N_PAD - 1, jnp.int32)]).reshape(-1, CH)
    ones = jnp.ones((CH, D), jnp.float32)
    zf = jnp.zeros((BB, D), jnp.float32)
    seq = jnp.arange(N_PAD, dtype=jnp.int32)
    psum, pdeg = _AGG(src2, dst2, x, ones, zf, seq)
    b = (b_self + b_neigh)[None, :]
    return _combine(x, psum, pdeg, W_self.T, W_neigh.T, b)


# speedup vs baseline: 1.7847x; 1.7847x over previous
import functools

import jax
import jax.numpy as jnp
from jax import lax
from jax.experimental import pallas as pl
from jax.experimental.pallas import tpu as pltpu
from jax.experimental.pallas import tpu_sc as plsc

N_NODES = 10000
D = 128
E = 320000

NC = 2
NS = 16
CH = 80
NCHUNK = E // CH
CH_PER_SC = NCHUNK // NC
CH_PER_TILE = CH_PER_SC // NS
N_PAD = 10240
RPT = N_PAD // NS
BB = 64
NBB = RPT // BB


def _make_agg():
    mesh = plsc.VectorSubcoreMesh(core_axis_name="c", subcore_axis_name="s")

    @functools.partial(
        pl.kernel,
        mesh=mesh,
        out_type=[
            jax.ShapeDtypeStruct((NC, N_PAD, D), jnp.float32),
            jax.ShapeDtypeStruct((NC, N_PAD, D), jnp.float32),
        ],
        scratch_types=[
            pltpu.VMEM((CH,), jnp.int32),
            pltpu.VMEM((CH,), jnp.int32),
            pltpu.VMEM((CH,), jnp.int32),
            pltpu.VMEM((CH,), jnp.int32),
            pltpu.VMEM((CH, D), jnp.float32),
            pltpu.VMEM((CH, D), jnp.float32),
            pltpu.VMEM((CH, D), jnp.float32),
            pltpu.VMEM((BB, D), jnp.float32),
            pltpu.VMEM((RPT,), jnp.int32),
            pltpu.VMEM((BB,), jnp.int32),
            pltpu.VMEM_SHARED((N_PAD, D), jnp.float32),
            pltpu.SemaphoreType.DMA,
            pltpu.SemaphoreType.DMA,
        ],
    )
    def agg(src_hbm, dst_hbm, x_hbm, ones_hbm, zf_hbm, seq_hbm,
            psum_hbm, pdeg_hbm,
            src_a, dst_a, src_b, dst_b, rows_a, rows_b, ones_v,
            buf_v, seq_v, seq1_v,
            sum_sh, sem, semb):
        cid = lax.axis_index("c")
        sid = lax.axis_index("s")
        nb = sid * RPT

        pltpu.sync_copy(zf_hbm, buf_v)
        pltpu.sync_copy(ones_hbm, ones_v)
        pltpu.sync_copy(seq_hbm.at[pl.ds(nb, RPT)], seq_v)

        def zinit(t, carry):
            pltpu.sync_copy(seq_hbm.at[pl.ds(nb + t * BB, BB)], seq1_v)
            pltpu.sync_copy(buf_v, sum_sh.at[seq1_v])
            return carry

        lax.fori_loop(0, NBB, zinit, 0)
        plsc.subcore_barrier()

        base_chunk = cid * CH_PER_SC + sid

        def eoff(k):
            return (base_chunk + jnp.minimum(k, CH_PER_TILE - 1) * NS) * CH

        def start(k, src_x, rows_x, sem_x):
            e0 = eoff(k)
            pltpu.sync_copy(src_hbm.at[pl.ds(e0, CH)], src_x)
            pltpu.async_copy(x_hbm.at[src_x], rows_x, sem_x)

        def ldst(k, dst_x):
            pltpu.sync_copy(dst_hbm.at[pl.ds(eoff(k), CH)], dst_x)

        def drain(rows_x, sem_x):
            pltpu.make_async_copy(x_hbm.at[pl.ds(0, CH)], rows_x, sem_x).wait()

        ldst(0, dst_a)
        start(0, src_a, rows_a, sem)
        ldst(1, dst_b)
        start(1, src_b, rows_b, semb)

        def body(q, carry):
            c = 2 * q
            drain(rows_a, sem)
            pltpu.sync_copy(rows_a, sum_sh.at[dst_a], add=True)
            ldst(c + 2, dst_a)
            start(c + 2, src_a, rows_a, sem)
            drain(rows_b, semb)
            pltpu.sync_copy(rows_b, sum_sh.at[dst_b], add=True)
            ldst(c + 3, dst_b)
            start(c + 3, src_b, rows_b, semb)
            return carry

        lax.fori_loop(0, (CH_PER_TILE - 1) // 2, body, 0)
        drain(rows_a, sem)
        pltpu.sync_copy(rows_a, sum_sh.at[dst_a], add=True)
        drain(rows_b, semb)
        plsc.subcore_barrier()

        def wout_sum(t, carry):
            r0 = nb + t * BB
            idx = seq_v.at[pl.ds(t * BB, BB)]
            pltpu.async_copy(sum_sh.at[idx], buf_v, sem).wait()
            pltpu.sync_copy(buf_v, psum_hbm.at[cid, pl.ds(r0, BB)])
            return carry

        lax.fori_loop(0, NBB, wout_sum, 0)
        plsc.subcore_barrier()

        pltpu.sync_copy(zf_hbm, buf_v)
        lax.fori_loop(0, NBB, zinit, 0)
        plsc.subcore_barrier()

        ldst(0, dst_a)

        def body2(q, carry):
            c = 2 * q
            ldst(c + 1, dst_b)
            pltpu.sync_copy(ones_v, sum_sh.at[dst_a], add=True)
            ldst(c + 2, dst_a)
            pltpu.sync_copy(ones_v, sum_sh.at[dst_b], add=True)
            return carry

        lax.fori_loop(0, (CH_PER_TILE - 1) // 2, body2, 0)
        pltpu.sync_copy(ones_v, sum_sh.at[dst_a], add=True)
        plsc.subcore_barrier()

        def wout_deg(t, carry):
            r0 = nb + t * BB
            idx = seq_v.at[pl.ds(t * BB, BB)]
            pltpu.async_copy(sum_sh.at[idx], buf_v, sem).wait()
            pltpu.sync_copy(buf_v, pdeg_hbm.at[cid, pl.ds(r0, BB)])
            return carry

        lax.fori_loop(0, NBB, wout_deg, 0)

    return agg


_AGG = _make_agg()

BN = 1000


def _tc_body(x_ref, ps_ref, pd_ref, wst_ref, wnt_ref, b_ref, o_ref):
    s = ps_ref[0] + ps_ref[1]
    deg = pd_ref[0, :, :1] + pd_ref[1, :, :1]
    h = s / jnp.maximum(deg, 1.0)
    acc = jnp.dot(x_ref[...], wst_ref[...], preferred_element_type=jnp.float32)
    acc = acc + jnp.dot(h, wnt_ref[...], preferred_element_type=jnp.float32)
    o_ref[...] = acc + b_ref[...]


def _combine(x, psum, pdeg, wst, wnt, b):
    return pl.pallas_call(
        _tc_body,
        grid=(N_NODES // BN,),
        in_specs=[
            pl.BlockSpec((BN, D), lambda i: (i, 0)),
            pl.BlockSpec((NC, BN, D), lambda i: (0, i, 0)),
            pl.BlockSpec((NC, BN, D), lambda i: (0, i, 0)),
            pl.BlockSpec((D, D), lambda i: (0, 0)),
            pl.BlockSpec((D, D), lambda i: (0, 0)),
            pl.BlockSpec((1, D), lambda i: (0, 0)),
        ],
        out_specs=pl.BlockSpec((BN, D), lambda i: (i, 0)),
        out_shape=jax.ShapeDtypeStruct((N_NODES, D), jnp.float32),
    )(x, psum, pdeg, wst, wnt, b)


def kernel(act_flag, x, edge_index, W_self, b_self, W_neigh, b_neigh):
    src = edge_index[0].astype(jnp.int32)
    dst = edge_index[1].astype(jnp.int32)
    ones = jnp.ones((CH, D), jnp.float32)
    zf = jnp.zeros((BB, D), jnp.float32)
    seq = jnp.arange(N_PAD, dtype=jnp.int32)
    psum, pdeg = _AGG(src, dst, x, ones, zf, seq)
    b = (b_self + b_neigh)[None, :]
    return _combine(x, psum, pdeg, W_self.T, W_neigh.T, b)


# speedup vs baseline: 2.0781x; 1.1644x over previous
import functools

import jax
import jax.numpy as jnp
from jax import lax
from jax.experimental import pallas as pl
from jax.experimental.pallas import tpu as pltpu
from jax.experimental.pallas import tpu_sc as plsc

N_NODES = 10000
D = 128
E = 320000

NC = 2
NS = 16
CH = 80
NCHUNK = E // CH
CH_PER_SC = NCHUNK // NC
CH_PER_TILE = CH_PER_SC // NS
N_PAD = 10240
RPT = N_PAD // NS
BB = 64
NBB = RPT // BB


def _make_agg():
    mesh = plsc.VectorSubcoreMesh(core_axis_name="c", subcore_axis_name="s")

    @functools.partial(
        pl.kernel,
        mesh=mesh,
        out_type=[
            jax.ShapeDtypeStruct((NC, N_PAD, D), jnp.float32),
            jax.ShapeDtypeStruct((NC, N_PAD, D), jnp.float32),
        ],
        scratch_types=[
            pltpu.VMEM((CH,), jnp.int32),
            pltpu.VMEM((CH,), jnp.int32),
            pltpu.VMEM((CH,), jnp.int32),
            pltpu.VMEM((CH,), jnp.int32),
            pltpu.VMEM((CH, D), jnp.float32),
            pltpu.VMEM((CH, D), jnp.float32),
            pltpu.VMEM((CH, D), jnp.float32),
            pltpu.VMEM((BB, D), jnp.float32),
            pltpu.VMEM((RPT,), jnp.int32),
            pltpu.VMEM((BB,), jnp.int32),
            pltpu.VMEM_SHARED((N_PAD, D), jnp.float32),
            pltpu.SemaphoreType.DMA,
            pltpu.SemaphoreType.DMA,
        ],
    )
    def agg(src_hbm, dst_hbm, x_hbm, ones_hbm, zf_hbm, seq_hbm,
            psum_hbm, pdeg_hbm,
            src_a, dst_a, src_b, dst_b, rows_a, rows_b, ones_v,
            buf_v, seq_v, seq1_v,
            sum_sh, sem, semb):
        cid = lax.axis_index("c")
        sid = lax.axis_index("s")
        nb = sid * RPT

        pltpu.sync_copy(zf_hbm, buf_v)
        pltpu.sync_copy(ones_hbm, ones_v)
        pltpu.sync_copy(seq_hbm.at[pl.ds(nb, RPT)], seq_v)

        def zinit(t, carry):
            pltpu.sync_copy(seq_hbm.at[pl.ds(nb + t * BB, BB)], seq1_v)
            pltpu.sync_copy(buf_v, sum_sh.at[seq1_v])
            return carry

        lax.fori_loop(0, NBB, zinit, 0)
        plsc.subcore_barrier()

        base_chunk = cid * CH_PER_SC + sid

        def eoff(k):
            return (base_chunk + jnp.minimum(k, CH_PER_TILE - 1) * NS) * CH

        def start(k, src_x, rows_x, sem_x):
            e0 = eoff(k)
            pltpu.sync_copy(src_hbm.at[pl.ds(e0, CH)], src_x)
            pltpu.async_copy(x_hbm.at[src_x], rows_x, sem_x)

        def ldst(k, dst_x):
            pltpu.sync_copy(dst_hbm.at[pl.ds(eoff(k), CH)], dst_x)

        def drain(rows_x, sem_x):
            pltpu.make_async_copy(x_hbm.at[pl.ds(0, CH)], rows_x, sem_x).wait()

        ldst(0, dst_a)
        start(0, src_a, rows_a, sem)
        ldst(1, dst_b)
        start(1, src_b, rows_b, semb)

        def body(q, carry):
            c = 2 * q
            drain(rows_a, sem)
            pltpu.sync_copy(rows_a, sum_sh.at[dst_a], add=True)
            ldst(c + 2, dst_a)
            start(c + 2, src_a, rows_a, sem)
            drain(rows_b, semb)
            pltpu.sync_copy(rows_b, sum_sh.at[dst_b], add=True)
            ldst(c + 3, dst_b)
            start(c + 3, src_b, rows_b, semb)
            return carry

        lax.fori_loop(0, (CH_PER_TILE - 1) // 2, body, 0)
        drain(rows_a, sem)
        pltpu.sync_copy(rows_a, sum_sh.at[dst_a], add=True)
        drain(rows_b, semb)
        plsc.subcore_barrier()

        def wout_sum(t, carry):
            r0 = nb + t * BB
            idx = seq_v.at[pl.ds(t * BB, BB)]
            pltpu.async_copy(sum_sh.at[idx], buf_v, sem).wait()
            pltpu.sync_copy(buf_v, psum_hbm.at[cid, pl.ds(r0, BB)])
            return carry

        lax.fori_loop(0, NBB, wout_sum, 0)
        plsc.subcore_barrier()

        pltpu.sync_copy(zf_hbm, buf_v)
        lax.fori_loop(0, NBB, zinit, 0)
        plsc.subcore_barrier()

        def drain2(sem_x):
            pltpu.make_async_copy(ones_v, sum_sh.at[dst_a], sem_x).wait()

        ldst(0, dst_a)
        pltpu.async_copy(ones_v, sum_sh.at[dst_a], sem, add=True)
        ldst(1, dst_b)
        pltpu.async_copy(ones_v, sum_sh.at[dst_b], semb, add=True)

        def body2(q, carry):
            c = 2 * q
            drain2(sem)
            ldst(c + 2, dst_a)
            pltpu.async_copy(ones_v, sum_sh.at[dst_a], sem, add=True)
            drain2(semb)
            ldst(c + 3, dst_b)
            pltpu.async_copy(ones_v, sum_sh.at[dst_b], semb, add=True)
            return carry

        lax.fori_loop(0, (CH_PER_TILE - 3) // 2, body2, 0)
        drain2(sem)
        ldst(CH_PER_TILE - 1, dst_a)
        pltpu.async_copy(ones_v, sum_sh.at[dst_a], sem, add=True)
        drain2(semb)
        drain2(sem)
        plsc.subcore_barrier()

        def wout_deg(t, carry):
            r0 = nb + t * BB
            idx = seq_v.at[pl.ds(t * BB, BB)]
            pltpu.async_copy(sum_sh.at[idx], buf_v, sem).wait()
            pltpu.sync_copy(buf_v, pdeg_hbm.at[cid, pl.ds(r0, BB)])
            return carry

        lax.fori_loop(0, NBB, wout_deg, 0)

    return agg


_AGG = _make_agg()

BN = 1000


def _tc_body(x_ref, ps_ref, pd_ref, wst_ref, wnt_ref, b_ref, o_ref):
    s = ps_ref[0] + ps_ref[1]
    deg = pd_ref[0, :, :1] + pd_ref[1, :, :1]
    h = s / jnp.maximum(deg, 1.0)
    acc = jnp.dot(x_ref[...], wst_ref[...], preferred_element_type=jnp.float32)
    acc = acc + jnp.dot(h, wnt_ref[...], preferred_element_type=jnp.float32)
    o_ref[...] = acc + b_ref[...]


def _combine(x, psum, pdeg, wst, wnt, b):
    return pl.pallas_call(
        _tc_body,
        grid=(N_NODES // BN,),
        in_specs=[
            pl.BlockSpec((BN, D), lambda i: (i, 0)),
            pl.BlockSpec((NC, BN, D), lambda i: (0, i, 0)),
            pl.BlockSpec((NC, BN, D), lambda i: (0, i, 0)),
            pl.BlockSpec((D, D), lambda i: (0, 0)),
            pl.BlockSpec((D, D), lambda i: (0, 0)),
            pl.BlockSpec((1, D), lambda i: (0, 0)),
        ],
        out_specs=pl.BlockSpec((BN, D), lambda i: (i, 0)),
        out_shape=jax.ShapeDtypeStruct((N_NODES, D), jnp.float32),
    )(x, psum, pdeg, wst, wnt, b)


def kernel(act_flag, x, edge_index, W_self, b_self, W_neigh, b_neigh):
    src = edge_index[0].astype(jnp.int32)
    dst = edge_index[1].astype(jnp.int32)
    ones = jnp.ones((CH, D), jnp.float32)
    zf = jnp.zeros((BB, D), jnp.float32)
    seq = jnp.arange(N_PAD, dtype=jnp.int32)
    psum, pdeg = _AGG(src, dst, x, ones, zf, seq)
    b = (b_self + b_neigh)[None, :]
    return _combine(x, psum, pdeg, W_self.T, W_neigh.T, b)
